# Initial kernel scaffold; baseline (speedup 1.0000x reference)
import jax
import jax.numpy as jnp
from jax.experimental import pallas as pl


def kernel(x, edge_index, batch, eps_ld, eps_ud, params_ld, params_ud):
    raise NotImplementedError("write your pallas kernel here")


# speedup vs baseline: 5.6045x; 5.6045x over previous
import functools

import jax
import jax.numpy as jnp
from jax import lax
from jax.experimental import pallas as pl
from jax.experimental.pallas import tpu as pltpu, tpu_sc as plsc

N = 16384
G = 32
NPG = 512
E = 262144
EPG = E // G
D = 128
H = 16
L = 8
NL = 10
ECH = 1024
NPART = 2

_BOUND_OFF_128 = (None, 208, 416, 624, 592, 560, 528, 496, 464, 432,
                  400, 368, 336, 304, 272, 240)
_BOUND_OFF_16 = (None,) + tuple(16 * k for k in range(1, 16))


def _graph_bounds(offs):
    return tuple(
        (g + 1) * EPG if offs[g % 16] is None else g * EPG + offs[g % 16]
        for g in range(G))


def _agg_sc_body(wd, h_hbm, src_hbm, dst_hbm, part_hbm, zeros_hbm, out_hbm,
                 h_v, soff_v, doff_v, part_v, acc_v):
    c = lax.axis_index("c")
    s = lax.axis_index("s")
    g = s * 2 + c
    pltpu.sync_copy(h_hbm.at[g], h_v)
    pltpu.sync_copy(src_hbm.at[pl.ds(g * EPG, EPG)], soff_v)
    pltpu.sync_copy(dst_hbm.at[pl.ds(g * EPG, EPG)], doff_v)
    pltpu.sync_copy(part_hbm.at[pl.ds(g * EPG, EPG)], part_v)
    pltpu.sync_copy(zeros_hbm, acc_v)

    def obody(i, carry):
        sv = soff_v[pl.ds(i * 16, 16)]
        dv = doff_v[pl.ds(i * 16, 16)]
        p = part_v[pl.ds(i * 16, 16)]
        soff_v[pl.ds(i * 16, 16)] = (sv & (NPG - 1)) * wd
        doff_v[pl.ds(i * 16, 16)] = (p * NPG + (dv & (NPG - 1))) * wd
        return carry

    lax.fori_loop(0, EPG // 16, obody, 0)

    def ebody(i, carry):
        so = soff_v[pl.ds(i * 16, 16)]
        do = doff_v[pl.ds(i * 16, 16)]
        for j in range(16):
            sj = so[j]
            dj = do[j]
            for k in range(wd // 16):
                acc_v[pl.ds(dj + k * 16, 16)] += h_v[pl.ds(sj + k * 16, 16)]
        return carry

    lax.fori_loop(0, EPG // 16, ebody, 0)

    def mbody(i, carry):
        t = acc_v[pl.ds(i * 16, 16)]
        for p in range(1, NPART):
            t = t + acc_v[pl.ds(p * NPG * wd + i * 16, 16)]
        h_v[pl.ds(i * 16, 16)] = t
        return carry

    lax.fori_loop(0, NPG * wd // 16, mbody, 0)
    pltpu.sync_copy(h_v, out_hbm.at[g])


def _agg_sc(h, src, dst, part, wd):
    h2 = h.reshape(G, NPG * wd)
    zeros = jnp.zeros((NPART * NPG * wd,), jnp.float32)
    mesh = plsc.VectorSubcoreMesh(core_axis_name="c", subcore_axis_name="s",
                                  num_cores=2, num_subcores=16)
    out = pl.kernel(
        functools.partial(_agg_sc_body, wd),
        out_type=jax.ShapeDtypeStruct((G, NPG * wd), jnp.float32),
        mesh=mesh,
        scratch_types=[
            pltpu.VMEM((NPG * wd,), jnp.float32),
            pltpu.VMEM((EPG,), jnp.int32),
            pltpu.VMEM((EPG,), jnp.int32),
            pltpu.VMEM((EPG,), jnp.int32),
            pltpu.VMEM((NPART * NPG * wd,), jnp.float32),
        ],
    )(h2, src, dst, part, zeros)
    return out.reshape(N, wd)


MLP_BLK = 2048


def _mlp_tc_body(relu_out, h_ref, agg_ref, w1_ref, b1_ref, w2_ref, b2_ref,
                 out_ref):
    m = h_ref[...] + agg_ref[...]
    t = jax.lax.dot_general(m.astype(jnp.bfloat16), w1_ref[...],
                            (((1,), (0,)), ((), ())),
                            preferred_element_type=jnp.float32) + b1_ref[...]
    t = jnp.maximum(t, 0.0)
    u = jax.lax.dot_general(t.astype(jnp.bfloat16), w2_ref[...],
                            (((1,), (0,)), ((), ())),
                            preferred_element_type=jnp.float32) + b2_ref[...]
    if relu_out:
        u = jnp.maximum(u, 0.0)
    out_ref[...] = u


def _mlp_tc(h, agg, w1b, b1, w2b, b2, relu_out):
    n, win = h.shape
    wout = w2b.shape[1]
    return pl.pallas_call(
        functools.partial(_mlp_tc_body, relu_out),
        grid=(n // MLP_BLK,),
        in_specs=[
            pl.BlockSpec((MLP_BLK, win), lambda i: (i, 0)),
            pl.BlockSpec((MLP_BLK, win), lambda i: (i, 0)),
            pl.BlockSpec(w1b.shape, lambda i: (0, 0)),
            pl.BlockSpec((1, wout), lambda i: (0, 0)),
            pl.BlockSpec(w2b.shape, lambda i: (0, 0)),
            pl.BlockSpec((1, wout), lambda i: (0, 0)),
        ],
        out_specs=pl.BlockSpec((MLP_BLK, wout), lambda i: (i, 0)),
        out_shape=jax.ShapeDtypeStruct((n, wout), jnp.float32),
    )(h, agg, w1b, b1, w2b, b2)


def _final_tc_body(src_ref, dst_ref, h_ref, ecat_ref, whead_ref, bhead_ref,
                   nll_ref, kl_ref, sig_ref, adj_ref, a_acc):
    g = pl.program_id(0)

    srcl = src_ref[0] & (NPG - 1)
    dstl = dst_ref[0] & (NPG - 1)
    for c in range(EPG // ECH):
        s = srcl[:, c * ECH:(c + 1) * ECH].reshape(ECH, 1)
        d = dstl[:, c * ECH:(c + 1) * ECH].reshape(ECH, 1)
        col = jax.lax.broadcasted_iota(jnp.int32, (ECH, NPG), 1)
        oh_s = (s == col).astype(jnp.bfloat16)
        oh_d = (d == col).astype(jnp.bfloat16)
        part = jax.lax.dot_general(
            oh_s, oh_d, (((0,), (0,)), ((), ())),
            preferred_element_type=jnp.float32)
        if c == 0:
            a_acc[...] = part
        else:
            a_acc[...] += part
    a = a_acc[...]
    adj_ref[0] = a

    h = h_ref[0]
    head = jax.lax.dot_general(h.astype(jnp.bfloat16), whead_ref[...],
                               (((1,), (0,)), ((), ())),
                               preferred_element_type=jnp.float32) \
        + bhead_ref[...]
    mu = head[:, :2 * L]
    std = jax.nn.softplus(head[:, 2 * L:])
    z = mu + std * ecat_ref[0]
    zl = z[:, :L].astype(jnp.bfloat16)
    zu = z[:, L:].astype(jnp.bfloat16)
    ap = jax.lax.dot_general(zl, zu, (((1,), (1,)), ((), ())),
                             preferred_element_type=jnp.float32)
    sig_ref[0] = jax.nn.sigmoid(ap)
    nll_g = jnp.sum(jax.nn.softplus(ap)) - jnp.sum(a * ap)
    kl_g = jnp.sum(-jnp.log(std) + (std * std + mu * mu - 1.0) * 0.5)

    @pl.when(g == 0)
    def _():
        nll_ref[...] = nll_g.reshape(1, 1)
        kl_ref[...] = kl_g.reshape(1, 1)

    @pl.when(g > 0)
    def _():
        nll_ref[...] += nll_g.reshape(1, 1)
        kl_ref[...] += kl_g.reshape(1, 1)


def _final_tc(srcr, dstr, h, ecat, whead_b, bhead):
    return pl.pallas_call(
        _final_tc_body,
        grid=(G,),
        in_specs=[
            pl.BlockSpec((1, 1, EPG), lambda g: (g, 0, 0)),
            pl.BlockSpec((1, 1, EPG), lambda g: (g, 0, 0)),
            pl.BlockSpec((1, NPG, 2 * H), lambda g: (g, 0, 0)),
            pl.BlockSpec((1, NPG, 2 * L), lambda g: (g, 0, 0)),
            pl.BlockSpec((2 * H, 4 * L), lambda g: (0, 0)),
            pl.BlockSpec((1, 4 * L), lambda g: (0, 0)),
        ],
        out_specs=[
            pl.BlockSpec((1, 1), lambda g: (0, 0)),
            pl.BlockSpec((1, 1), lambda g: (0, 0)),
            pl.BlockSpec((1, NPG, NPG), lambda g: (g, 0, 0)),
            pl.BlockSpec((1, NPG, NPG), lambda g: (g, 0, 0)),
        ],
        out_shape=[
            jax.ShapeDtypeStruct((1, 1), jnp.float32),
            jax.ShapeDtypeStruct((1, 1), jnp.float32),
            jax.ShapeDtypeStruct((G, NPG, NPG), jnp.float32),
            jax.ShapeDtypeStruct((G, NPG, NPG), jnp.float32),
        ],
        scratch_shapes=[pltpu.VMEM((NPG, NPG), jnp.float32)],
    )(srcr, dstr, h.reshape(G, NPG, 2 * H), ecat, whead_b, bhead)


def _blockdiag(a, b):
    k, n = a.shape
    z = jnp.zeros((k, n), a.dtype)
    return jnp.concatenate(
        [jnp.concatenate([a, z], axis=1), jnp.concatenate([z, b], axis=1)],
        axis=0)


@jax.jit
def kernel(x, edge_index, batch, eps_ld, eps_ud, params_ld, params_ud):
    src = edge_index[0]
    dst = edge_index[1]

    order = jnp.argsort(dst, stable=True)
    pos = jnp.zeros((E,), jnp.int32).at[order].set(
        jnp.arange(E, dtype=jnp.int32))
    ge = jnp.arange(E, dtype=jnp.int32) // EPG
    part128 = (pos >= jnp.asarray(_graph_bounds(_BOUND_OFF_128),
                                  jnp.int32)[ge]).astype(jnp.int32)
    part16 = (pos >= jnp.asarray(_graph_bounds(_BOUND_OFF_16),
                                 jnp.int32)[ge]).astype(jnp.int32)

    ll, lu = params_ld["layers"], params_ud["layers"]
    w1c0 = jnp.concatenate([ll[0][0], lu[0][0]], axis=1).astype(jnp.bfloat16)
    b1c = [jnp.concatenate([ll[i][1], lu[i][1]]).reshape(1, 2 * H)
           for i in range(NL)]
    b2c = [jnp.concatenate([ll[i][3], lu[i][3]]).reshape(1, 2 * H)
           for i in range(NL)]
    w1bd = [_blockdiag(ll[i][0], lu[i][0]).astype(jnp.bfloat16)
            for i in range(1, NL)]
    w2bd = [_blockdiag(ll[i][2], lu[i][2]).astype(jnp.bfloat16)
            for i in range(NL)]
    whead = jnp.concatenate(
        [_blockdiag(params_ld["Wm"], params_ud["Wm"]),
         _blockdiag(params_ld["Ws"], params_ud["Ws"])],
        axis=1).astype(jnp.bfloat16)
    bhead = jnp.concatenate(
        [params_ld["bm"], params_ud["bm"],
         params_ld["bs"], params_ud["bs"]]).reshape(1, 4 * L)

    agg0a = _agg_sc(x[:, :D // 2], src, dst, part128, D // 2)
    agg0b = _agg_sc(x[:, D // 2:], src, dst, part128, D // 2)
    agg0 = jnp.concatenate([agg0a, agg0b], axis=1)
    h = _mlp_tc(x, agg0, w1c0, b1c[0], w2bd[0], b2c[0], relu_out=True)

    for l in range(1, NL):
        agg = _agg_sc(h, src, dst, part16, 2 * H)
        h = _mlp_tc(h, agg, w1bd[l - 1], b1c[l], w2bd[l], b2c[l],
                    relu_out=(l < NL - 1))

    ecat = jnp.concatenate([eps_ld.reshape(G, NPG, L),
                            eps_ud.reshape(G, NPG, L)], axis=2)
    srcr = src.reshape(G, 1, EPG)
    dstr = dst.reshape(G, 1, EPG)
    nll, kl, sig, adj = _final_tc(srcr, dstr, h, ecat, whead, bhead)
    return (nll[0, 0], kl[0, 0], sig, adj)
